# baseline (speedup 1.0000x reference)
import functools

import jax
import jax.numpy as jnp
from jax import lax
from jax.experimental import pallas as pl
from jax.experimental.pallas import tpu as pltpu
from jax.experimental.pallas import tpu_sc as plsc

SOM0, SOM1 = 32, 32
K = SOM0 * SOM1
LAT = 64
HID = 256
INP = 512
B = 1024
RB = 1024
NBLK = B // RB
SENTINEL = K

_F32 = jnp.float32
_HIGH = lax.Precision.HIGHEST


def _dot(a, b, trans_b=False, precision=None):
    dn = (((1,), (1 if trans_b else 0,)), ((), ()))
    return lax.dot_general(a, b, dimension_numbers=dn,
                           precision=precision, preferred_element_type=_F32)


def _sigmoid(v):
    return 1.0 / (1.0 + jnp.exp(-v))


def _tca_body(x_ref, e_ref, we1_ref, be1_ref, we2_ref, be2_ref,
              zet_ref, dist_ref, k_ref, idx8_ref, zq_ref):
    xb = x_ref[...]
    h = jnp.maximum(_dot(xb, we1_ref[...]) + be1_ref[...], 0.0)
    ze = jnp.maximum(_dot(h, we2_ref[...]) + be2_ref[...], 0.0)

    ze_t = jnp.maximum(
        lax.dot_general(we2_ref[...], h, dimension_numbers=(((0,), (1,)), ((), ())),
                        preferred_element_type=_F32) + be2_ref[...].reshape(LAT, 1),
        0.0)
    zet_ref[...] = ze_t

    e = e_ref[...]
    ze_hi = lax.convert_element_type(
        lax.convert_element_type(ze, jnp.bfloat16), _F32)
    ze_lo = ze - ze_hi
    eh = lax.convert_element_type(
        lax.convert_element_type(e, jnp.bfloat16), _F32)
    el = e - eh
    dots = (_dot(ze_hi, eh, trans_b=True) + _dot(ze_hi, el, trans_b=True)
            + _dot(ze_lo, eh, trans_b=True))
    e2 = jnp.sum(e * e, axis=1)
    ze2 = jnp.sum(ze * ze, axis=1, keepdims=True)
    dist = ze2 - 2.0 * dots + e2[None, :]
    dist_ref[...] = dist

    big = jnp.int32(2 ** 30)
    iota = lax.broadcasted_iota(jnp.int32, (RB, K), 1)
    m1 = jnp.min(dist, axis=1, keepdims=True)
    c1 = jnp.min(jnp.where(dist <= m1, iota, big), axis=1)
    masked = jnp.where(iota == c1[:, None], jnp.float32(jnp.inf), dist)
    m2 = jnp.min(masked, axis=1, keepdims=True)
    c2 = jnp.min(jnp.where(masked <= m2, iota, big), axis=1)
    oh1 = (iota == c1[:, None]).astype(_F32)
    oh2 = (iota == c2[:, None]).astype(_F32)
    e_hi = eh
    e_rest = el
    e_mid = lax.convert_element_type(
        lax.convert_element_type(e_rest, jnp.bfloat16), _F32)
    e_lo = e_rest - e_mid
    def _exact_gather(oh):
        return (_dot(oh, e_hi) + _dot(oh, e_mid)) + _dot(oh, e_lo)
    e1 = _exact_gather(oh1)
    e2r = _exact_gather(oh2)
    d1 = jnp.sum((ze - e1) ** 2, axis=1)
    d2 = jnp.sum((ze - e2r) ** 2, axis=1)
    take2 = (d2 < d1) | ((d2 == d1) & (c2 < c1))
    idx = jnp.where(take2, c2, c1)
    k_ref[...] = idx.reshape(1, 1, RB)

    zq_ref[...] = jnp.where(take2[:, None], e2r, e1)

    col = lax.broadcasted_iota(jnp.int32, (RB, 8), 1)
    kb = idx[:, None]
    k2c = lax.bitwise_and(kb, SOM1 - 1)
    sent = jnp.int32(SENTINEL)
    v = jnp.where(col == 1, jnp.where(kb < K - SOM1, kb + SOM1, sent), kb)
    v = jnp.where(col == 2, jnp.where(kb >= SOM1, kb - SOM1, sent), v)
    v = jnp.where(col == 3, jnp.where(k2c < SOM1 - 1, kb + 1, sent), v)
    v = jnp.where(col == 4, jnp.where(k2c > 0, kb - 1, sent), v)
    v = jnp.where(col >= 5, sent, v)
    idx8_ref[...] = v


def _tca(x, e_flat, W_e1, b_e1, W_e2, b_e2):
    full = lambda shape: pl.BlockSpec(shape, lambda i: (0,) * len(shape))
    return pl.pallas_call(
        _tca_body,
        grid=(NBLK,),
        in_specs=[
            pl.BlockSpec((RB, INP), lambda i: (i, 0)),
            full((K, LAT)),
            full((INP, HID)), full((1, HID)),
            full((HID, LAT)), full((1, LAT)),
        ],
        out_specs=[
            pl.BlockSpec((LAT, RB), lambda i: (0, i)),
            pl.BlockSpec((RB, K), lambda i: (i, 0)),
            pl.BlockSpec((1, 1, RB), lambda i: (i, 0, 0)),
            pl.BlockSpec((RB, 8), lambda i: (i, 0)),
            pl.BlockSpec((RB, LAT), lambda i: (i, 0)),
        ],
        out_shape=[
            jax.ShapeDtypeStruct((LAT, B), _F32),
            jax.ShapeDtypeStruct((B, K), _F32),
            jax.ShapeDtypeStruct((NBLK, 1, RB), jnp.int32),
            jax.ShapeDtypeStruct((B, 8), jnp.int32),
            jax.ShapeDtypeStruct((B, LAT), _F32),
        ],
    )(x, e_flat, W_e1, b_e1, W_e2, b_e2)


def _tcb_body(zet_ref, zq_ref, wq1_ref, bq1_ref, wq2_ref, bq2_ref,
              wd1_ref, bd1_ref, wd2_ref, bd2_ref, xq_ref, xe_ref):
    dn0 = (((0,), (0,)), ((), ()))
    hq = jnp.maximum(_dot(zq_ref[...], wq1_ref[...]) + bq1_ref[...], 0.0)
    xq_ref[...] = _sigmoid(_dot(hq, wq2_ref[...]) + bq2_ref[...])
    hd = jnp.maximum(
        lax.dot_general(zet_ref[...], wd1_ref[...], dimension_numbers=dn0,
                        preferred_element_type=_F32) + bd1_ref[...], 0.0)
    xe_ref[...] = _sigmoid(_dot(hd, wd2_ref[...]) + bd2_ref[...])


def _tcb(z_e_t, z_q, W_q1, b_q1, W_q2, b_q2, W_d1, b_d1, W_d2, b_d2):
    full = lambda shape: pl.BlockSpec(shape, lambda i: (0,) * len(shape))
    return pl.pallas_call(
        _tcb_body,
        grid=(NBLK,),
        in_specs=[
            pl.BlockSpec((LAT, RB), lambda i: (0, i)),
            pl.BlockSpec((RB, LAT), lambda i: (i, 0)),
            full((LAT, HID)), full((1, HID)),
            full((HID, INP)), full((1, INP)),
            full((LAT, HID)), full((1, HID)),
            full((HID, INP)), full((1, INP)),
        ],
        out_specs=[
            pl.BlockSpec((RB, INP), lambda i: (i, 0)),
            pl.BlockSpec((RB, INP), lambda i: (i, 0)),
        ],
        out_shape=[
            jax.ShapeDtypeStruct((B, INP), _F32),
            jax.ShapeDtypeStruct((B, INP), _F32),
        ],
    )(z_e_t, z_q, W_q1, b_q1, W_q2, b_q2, W_d1, b_d1, W_d2, b_d2)


def _sc_gather(idx5, table_pad):
    info = plsc.get_sparse_core_info()
    nc, ns = info.num_cores, info.num_subcores
    nw = nc * ns
    rows_w = B // nw
    nbr_h = rows_w * 5 // 2
    mesh = plsc.VectorSubcoreMesh(core_axis_name="c", subcore_axis_name="s")

    @functools.partial(
        pl.kernel,
        out_type=jax.ShapeDtypeStruct((B * 5, LAT), _F32),
        mesh=mesh,
        compiler_params=pltpu.CompilerParams(use_tc_tiling_on_sc=False),
        scratch_types=[
            pltpu.VMEM((2 * nbr_h,), jnp.int32),
            pltpu.VMEM((2 * nbr_h, LAT), _F32),
            pltpu.SemaphoreType.DMA,
            pltpu.SemaphoreType.DMA,
        ],
    )
    def body(idx5_hbm, tab_hbm, nbr_hbm, nidx, nrows, sem0, sem1):
        wid = lax.axis_index("s") * nc + lax.axis_index("c")
        base5 = wid * rows_w * 5
        pltpu.sync_copy(idx5_hbm.at[pl.ds(base5, 2 * nbr_h)], nidx)
        cp0 = pltpu.async_copy(tab_hbm.at[nidx.at[pl.ds(0, nbr_h)]],
                               nrows.at[pl.ds(0, nbr_h)], sem0)
        cp1 = pltpu.async_copy(tab_hbm.at[nidx.at[pl.ds(nbr_h, nbr_h)]],
                               nrows.at[pl.ds(nbr_h, nbr_h)], sem1)
        cp0.wait()
        cp1.wait()
        pltpu.sync_copy(nrows, nbr_hbm.at[pl.ds(base5, 2 * nbr_h)])

    return body(idx5, table_pad)


def kernel(x, embeddings, W_e1, b_e1, W_e2, b_e2, W_q1, b_q1, W_q2, b_q2,
           W_d1, b_d1, W_d2, b_d2):
    e_flat = embeddings.reshape(K, LAT)
    table_pad = jnp.concatenate([e_flat, jnp.zeros((8, LAT), _F32)], axis=0)

    (z_e_t, z_dist_flat, k_blk, idx8, z_q) = _tca(
        x, e_flat, W_e1, b_e1.reshape(1, HID), W_e2, b_e2.reshape(1, LAT))
    z_e = z_e_t.T
    k = k_blk.reshape(B)
    idx5 = idx8[:, :5].reshape(B * 5)

    nbr = _sc_gather(idx5, table_pad)
    z_q_neighbors = nbr.reshape(B, 5, LAT)

    x_hat_q, x_hat_e = _tcb(
        z_e_t, z_q, W_q1, b_q1.reshape(1, HID), W_q2, b_q2.reshape(1, INP),
        W_d1, b_d1.reshape(1, HID), W_d2, b_d2.reshape(1, INP))

    return (x_hat_q, x_hat_e, z_e, z_q, k, z_dist_flat, z_q_neighbors)


# speedup vs baseline: 1.0101x; 1.0101x over previous
import functools

import jax
import jax.numpy as jnp
from jax import lax
from jax.experimental import pallas as pl
from jax.experimental.pallas import tpu as pltpu
from jax.experimental.pallas import tpu_sc as plsc

SOM0, SOM1 = 32, 32
K = SOM0 * SOM1
LAT = 64
HID = 256
INP = 512
B = 1024
RB = 1024
NBLK = B // RB
SENTINEL = K

_F32 = jnp.float32
_HIGH = lax.Precision.HIGHEST


def _dot(a, b, trans_b=False, precision=None):
    dn = (((1,), (1 if trans_b else 0,)), ((), ()))
    return lax.dot_general(a, b, dimension_numbers=dn,
                           precision=precision, preferred_element_type=_F32)


def _sigmoid(v):
    return 1.0 / (1.0 + jnp.exp(-v))


def _tc_body(x_ref, e_ref, we1_ref, be1_ref, we2_ref, be2_ref,
             wq1_ref, bq1_ref, wq2_ref, bq2_ref,
             wd1_ref, bd1_ref, wd2_ref, bd2_ref,
             zet_ref, dist_ref, k_ref, idx8_ref, zq_ref, xq_ref, xe_ref):
    xb = x_ref[...]
    h = jnp.maximum(_dot(xb, we1_ref[...]) + be1_ref[...], 0.0)
    ze = jnp.maximum(_dot(h, we2_ref[...]) + be2_ref[...], 0.0)

    ze_t = jnp.maximum(
        lax.dot_general(we2_ref[...], h, dimension_numbers=(((0,), (1,)), ((), ())),
                        preferred_element_type=_F32) + be2_ref[...].reshape(LAT, 1),
        0.0)
    zet_ref[...] = ze_t

    e = e_ref[...]
    ze_hi = lax.convert_element_type(
        lax.convert_element_type(ze, jnp.bfloat16), _F32)
    ze_lo = ze - ze_hi
    eh = lax.convert_element_type(
        lax.convert_element_type(e, jnp.bfloat16), _F32)
    el = e - eh
    dots = (_dot(ze_hi, eh, trans_b=True) + _dot(ze_hi, el, trans_b=True)
            + _dot(ze_lo, eh, trans_b=True))
    e2 = jnp.sum(e * e, axis=1)
    ze2 = jnp.sum(ze * ze, axis=1, keepdims=True)
    dist = ze2 - 2.0 * dots + e2[None, :]
    dist_ref[...] = dist

    big = jnp.int32(2 ** 30)
    iota = lax.broadcasted_iota(jnp.int32, (RB, K), 1)
    m1 = jnp.min(dist, axis=1, keepdims=True)
    c1 = jnp.min(jnp.where(dist <= m1, iota, big), axis=1)
    masked = jnp.where(iota == c1[:, None], jnp.float32(jnp.inf), dist)
    m2 = jnp.min(masked, axis=1, keepdims=True)
    c2 = jnp.min(jnp.where(masked <= m2, iota, big), axis=1)
    oh1 = (iota == c1[:, None]).astype(_F32)
    oh2 = (iota == c2[:, None]).astype(_F32)
    e_hi = eh
    e_rest = el
    e_mid = lax.convert_element_type(
        lax.convert_element_type(e_rest, jnp.bfloat16), _F32)
    e_lo = e_rest - e_mid
    def _exact_gather(oh):
        return (_dot(oh, e_hi) + _dot(oh, e_mid)) + _dot(oh, e_lo)
    e1 = _exact_gather(oh1)
    e2r = _exact_gather(oh2)
    d1 = jnp.sum((ze - e1) ** 2, axis=1)
    d2 = jnp.sum((ze - e2r) ** 2, axis=1)
    take2 = (d2 < d1) | ((d2 == d1) & (c2 < c1))
    idx = jnp.where(take2, c2, c1)
    k_ref[...] = idx.reshape(1, 1, RB)

    zq = jnp.where(take2[:, None], e2r, e1)
    zq_ref[...] = zq

    col = lax.broadcasted_iota(jnp.int32, (RB, 8), 1)
    kb = idx[:, None]
    k2c = lax.bitwise_and(kb, SOM1 - 1)
    sent = jnp.int32(SENTINEL)
    v = jnp.where(col == 1, jnp.where(kb < K - SOM1, kb + SOM1, sent), kb)
    v = jnp.where(col == 2, jnp.where(kb >= SOM1, kb - SOM1, sent), v)
    v = jnp.where(col == 3, jnp.where(k2c < SOM1 - 1, kb + 1, sent), v)
    v = jnp.where(col == 4, jnp.where(k2c > 0, kb - 1, sent), v)
    v = jnp.where(col >= 5, sent, v)
    idx8_ref[...] = v

    hq = jnp.maximum(_dot(zq, wq1_ref[...]) + bq1_ref[...], 0.0)
    xq_ref[...] = _sigmoid(_dot(hq, wq2_ref[...]) + bq2_ref[...])
    hd = jnp.maximum(_dot(ze, wd1_ref[...]) + bd1_ref[...], 0.0)
    xe_ref[...] = _sigmoid(_dot(hd, wd2_ref[...]) + bd2_ref[...])


def _tc(x, e_flat, W_e1, b_e1, W_e2, b_e2, W_q1, b_q1, W_q2, b_q2,
        W_d1, b_d1, W_d2, b_d2):
    full = lambda shape: pl.BlockSpec(shape, lambda i: (0,) * len(shape))
    return pl.pallas_call(
        _tc_body,
        grid=(NBLK,),
        in_specs=[
            pl.BlockSpec((RB, INP), lambda i: (i, 0)),
            full((K, LAT)),
            full((INP, HID)), full((1, HID)),
            full((HID, LAT)), full((1, LAT)),
            full((LAT, HID)), full((1, HID)),
            full((HID, INP)), full((1, INP)),
            full((LAT, HID)), full((1, HID)),
            full((HID, INP)), full((1, INP)),
        ],
        out_specs=[
            pl.BlockSpec((LAT, RB), lambda i: (0, i)),
            pl.BlockSpec((RB, K), lambda i: (i, 0)),
            pl.BlockSpec((1, 1, RB), lambda i: (i, 0, 0)),
            pl.BlockSpec((RB, 8), lambda i: (i, 0)),
            pl.BlockSpec((RB, LAT), lambda i: (i, 0)),
            pl.BlockSpec((RB, INP), lambda i: (i, 0)),
            pl.BlockSpec((RB, INP), lambda i: (i, 0)),
        ],
        out_shape=[
            jax.ShapeDtypeStruct((LAT, B), _F32),
            jax.ShapeDtypeStruct((B, K), _F32),
            jax.ShapeDtypeStruct((NBLK, 1, RB), jnp.int32),
            jax.ShapeDtypeStruct((B, 8), jnp.int32),
            jax.ShapeDtypeStruct((B, LAT), _F32),
            jax.ShapeDtypeStruct((B, INP), _F32),
            jax.ShapeDtypeStruct((B, INP), _F32),
        ],
    )(x, e_flat, W_e1, b_e1, W_e2, b_e2, W_q1, b_q1, W_q2, b_q2,
      W_d1, b_d1, W_d2, b_d2)


def _sc_gather(idx5, table_pad):
    info = plsc.get_sparse_core_info()
    nc, ns = info.num_cores, info.num_subcores
    nw = nc * ns
    rows_w = B // nw
    nbr_h = rows_w * 5 // 2
    mesh = plsc.VectorSubcoreMesh(core_axis_name="c", subcore_axis_name="s")

    @functools.partial(
        pl.kernel,
        out_type=jax.ShapeDtypeStruct((B * 5, LAT), _F32),
        mesh=mesh,
        compiler_params=pltpu.CompilerParams(use_tc_tiling_on_sc=False, skip_device_barrier=True),
        scratch_types=[
            pltpu.VMEM((2 * nbr_h,), jnp.int32),
            pltpu.VMEM((2 * nbr_h, LAT), _F32),
            pltpu.SemaphoreType.DMA,
            pltpu.SemaphoreType.DMA,
        ],
    )
    def body(idx5_hbm, tab_hbm, nbr_hbm, nidx, nrows, sem0, sem1):
        wid = lax.axis_index("s") * nc + lax.axis_index("c")
        base5 = wid * rows_w * 5
        pltpu.sync_copy(idx5_hbm.at[pl.ds(base5, 2 * nbr_h)], nidx)
        cp0 = pltpu.async_copy(tab_hbm.at[nidx.at[pl.ds(0, nbr_h)]],
                               nrows.at[pl.ds(0, nbr_h)], sem0)
        cp1 = pltpu.async_copy(tab_hbm.at[nidx.at[pl.ds(nbr_h, nbr_h)]],
                               nrows.at[pl.ds(nbr_h, nbr_h)], sem1)
        cp0.wait()
        cp1.wait()
        pltpu.sync_copy(nrows, nbr_hbm.at[pl.ds(base5, 2 * nbr_h)])

    return body(idx5, table_pad)


def kernel(x, embeddings, W_e1, b_e1, W_e2, b_e2, W_q1, b_q1, W_q2, b_q2,
           W_d1, b_d1, W_d2, b_d2):
    e_flat = embeddings.reshape(K, LAT)
    table_pad = jnp.concatenate([e_flat, jnp.zeros((8, LAT), _F32)], axis=0)

    (z_e_t, z_dist_flat, k_blk, idx8, z_q, x_hat_q, x_hat_e) = _tc(
        x, e_flat, W_e1, b_e1.reshape(1, HID), W_e2, b_e2.reshape(1, LAT),
        W_q1, b_q1.reshape(1, HID), W_q2, b_q2.reshape(1, INP),
        W_d1, b_d1.reshape(1, HID), W_d2, b_d2.reshape(1, INP))
    z_e = z_e_t.T
    k = k_blk.reshape(B)
    idx5 = idx8[:, :5].reshape(B * 5)

    nbr = _sc_gather(idx5, table_pad)
    z_q_neighbors = nbr.reshape(B, 5, LAT)

    return (x_hat_q, x_hat_e, z_e, z_q, k, z_dist_flat, z_q_neighbors)


# speedup vs baseline: 1.0514x; 1.0409x over previous
import functools

import jax
import jax.numpy as jnp
from jax import lax
from jax.experimental import pallas as pl
from jax.experimental.pallas import tpu as pltpu
from jax.experimental.pallas import tpu_sc as plsc

SOM0, SOM1 = 32, 32
K = SOM0 * SOM1
LAT = 64
HID = 256
INP = 512
B = 1024
RB = 1024
NBLK = B // RB
SENTINEL = K

_F32 = jnp.float32
_HIGH = lax.Precision.HIGHEST


def _dot(a, b, trans_b=False, precision=None):
    dn = (((1,), (1 if trans_b else 0,)), ((), ()))
    return lax.dot_general(a, b, dimension_numbers=dn,
                           precision=precision, preferred_element_type=_F32)


def _sigmoid(v):
    return 1.0 / (1.0 + jnp.exp(-v))


def _tc_body(x_ref, e_ref, we1_ref, be1_ref, we2_ref, be2_ref,
             wq1_ref, bq1_ref, wq2_ref, bq2_ref,
             wd1_ref, bd1_ref, wd2_ref, bd2_ref,
             zet_ref, dist_ref, k_ref, idx8_ref, zq_ref, xq_ref, xe_ref):
    xb = x_ref[...]
    h = jnp.maximum(_dot(xb, we1_ref[...]) + be1_ref[...], 0.0)
    ze = jnp.maximum(_dot(h, we2_ref[...]) + be2_ref[...], 0.0)

    ze_t = jnp.maximum(
        lax.dot_general(we2_ref[...], h, dimension_numbers=(((0,), (1,)), ((), ())),
                        preferred_element_type=_F32) + be2_ref[...].reshape(LAT, 1),
        0.0)
    zet_ref[...] = ze_t

    e = e_ref[...]
    ze_hi = lax.convert_element_type(
        lax.convert_element_type(ze, jnp.bfloat16), _F32)
    ze_lo = ze - ze_hi
    eh = lax.convert_element_type(
        lax.convert_element_type(e, jnp.bfloat16), _F32)
    el = e - eh
    dots = (_dot(ze_hi, eh, trans_b=True) + _dot(ze_hi, el, trans_b=True)
            + _dot(ze_lo, eh, trans_b=True))
    e2 = jnp.sum(e * e, axis=1)
    ze2 = jnp.sum(ze * ze, axis=1, keepdims=True)
    dist = ze2 - 2.0 * dots + e2[None, :]
    dist_ref[...] = dist

    big = jnp.int32(2 ** 30)
    iota = lax.broadcasted_iota(jnp.int32, (RB, K), 1)
    m1 = jnp.min(dist, axis=1, keepdims=True)
    c1 = jnp.min(jnp.where(dist <= m1, iota, big), axis=1)
    masked = jnp.where(iota == c1[:, None], jnp.float32(jnp.inf), dist)
    m2 = jnp.min(masked, axis=1, keepdims=True)
    c2 = jnp.min(jnp.where(masked <= m2, iota, big), axis=1)
    oh1 = (iota == c1[:, None]).astype(_F32)
    oh2 = (iota == c2[:, None]).astype(_F32)
    e_hi = eh
    e_rest = el
    e_mid = lax.convert_element_type(
        lax.convert_element_type(e_rest, jnp.bfloat16), _F32)
    e_lo = e_rest - e_mid
    def _exact_gather(oh):
        return (_dot(oh, e_hi) + _dot(oh, e_mid)) + _dot(oh, e_lo)
    e1 = _exact_gather(oh1)
    e2r = _exact_gather(oh2)
    d1 = jnp.sum((ze - e1) ** 2, axis=1)
    d2 = jnp.sum((ze - e2r) ** 2, axis=1)
    take2 = (d2 < d1) | ((d2 == d1) & (c2 < c1))
    idx = jnp.where(take2, c2, c1)
    k_ref[...] = idx.reshape(1, 1, RB)

    zq = jnp.where(take2[:, None], e2r, e1)
    zq_ref[...] = zq

    col = lax.broadcasted_iota(jnp.int32, (RB, 8), 1)
    kb = idx[:, None]
    k2c = lax.bitwise_and(kb, SOM1 - 1)
    sent = jnp.int32(SENTINEL)
    v = jnp.where(col == 1, jnp.where(kb < K - SOM1, kb + SOM1, sent), kb)
    v = jnp.where(col == 2, jnp.where(kb >= SOM1, kb - SOM1, sent), v)
    v = jnp.where(col == 3, jnp.where(k2c < SOM1 - 1, kb + 1, sent), v)
    v = jnp.where(col == 4, jnp.where(k2c > 0, kb - 1, sent), v)
    v = jnp.where(col >= 5, sent, v)
    idx8_ref[...] = v

    hq = jnp.maximum(_dot(zq, wq1_ref[...]) + bq1_ref[...], 0.0)
    xq_ref[...] = _sigmoid(_dot(hq, wq2_ref[...]) + bq2_ref[...])
    hd = jnp.maximum(_dot(ze, wd1_ref[...]) + bd1_ref[...], 0.0)
    xe_ref[...] = _sigmoid(_dot(hd, wd2_ref[...]) + bd2_ref[...])


def _tc(x, e_flat, W_e1, b_e1, W_e2, b_e2, W_q1, b_q1, W_q2, b_q2,
        W_d1, b_d1, W_d2, b_d2):
    full = lambda shape: pl.BlockSpec(shape, lambda i: (0,) * len(shape))
    return pl.pallas_call(
        _tc_body,
        grid=(NBLK,),
        in_specs=[
            pl.BlockSpec((RB, INP), lambda i: (i, 0)),
            full((K, LAT)),
            full((INP, HID)), full((1, HID)),
            full((HID, LAT)), full((1, LAT)),
            full((LAT, HID)), full((1, HID)),
            full((HID, INP)), full((1, INP)),
            full((LAT, HID)), full((1, HID)),
            full((HID, INP)), full((1, INP)),
        ],
        out_specs=[
            pl.BlockSpec((LAT, RB), lambda i: (0, i)),
            pl.BlockSpec((RB, K), lambda i: (i, 0)),
            pl.BlockSpec((1, 1, RB), lambda i: (i, 0, 0)),
            pl.BlockSpec((RB, 8), lambda i: (i, 0)),
            pl.BlockSpec((RB, LAT), lambda i: (i, 0)),
            pl.BlockSpec((RB, INP), lambda i: (i, 0)),
            pl.BlockSpec((RB, INP), lambda i: (i, 0)),
        ],
        out_shape=[
            jax.ShapeDtypeStruct((LAT, B), _F32),
            jax.ShapeDtypeStruct((B, K), _F32),
            jax.ShapeDtypeStruct((NBLK, 1, RB), jnp.int32),
            jax.ShapeDtypeStruct((B, 8), jnp.int32),
            jax.ShapeDtypeStruct((B, LAT), _F32),
            jax.ShapeDtypeStruct((B, INP), _F32),
            jax.ShapeDtypeStruct((B, INP), _F32),
        ],
    )(x, e_flat, W_e1, b_e1, W_e2, b_e2, W_q1, b_q1, W_q2, b_q2,
      W_d1, b_d1, W_d2, b_d2)


def _sc_gather(idx5, table_pad):
    info = plsc.get_sparse_core_info()
    nc, ns = 1, info.num_subcores
    nw = nc * ns
    rows_w = B // nw
    nbr_h = rows_w * 5 // 4
    mesh = plsc.VectorSubcoreMesh(core_axis_name="c", subcore_axis_name="s",
                                  num_cores=1)

    @functools.partial(
        pl.kernel,
        out_type=jax.ShapeDtypeStruct((B * 5, LAT), _F32),
        mesh=mesh,
        compiler_params=pltpu.CompilerParams(use_tc_tiling_on_sc=False),
        scratch_types=[
            pltpu.VMEM((4 * nbr_h,), jnp.int32),
            pltpu.VMEM((4 * nbr_h, LAT), _F32),
            pltpu.SemaphoreType.DMA,
            pltpu.SemaphoreType.DMA,
            pltpu.SemaphoreType.DMA,
            pltpu.SemaphoreType.DMA,
        ],
    )
    def body(idx5_hbm, tab_hbm, nbr_hbm, nidx, nrows, sem0, sem1, sem2, sem3):
        wid = lax.axis_index("s")
        base5 = wid * rows_w * 5
        pltpu.sync_copy(idx5_hbm.at[pl.ds(base5, 4 * nbr_h)], nidx)
        cps = []
        for j, sem in enumerate((sem0, sem1, sem2, sem3)):
            cps.append(pltpu.async_copy(
                tab_hbm.at[nidx.at[pl.ds(j * nbr_h, nbr_h)]],
                nrows.at[pl.ds(j * nbr_h, nbr_h)], sem))
        for cp in cps:
            cp.wait()
        pltpu.sync_copy(nrows, nbr_hbm.at[pl.ds(base5, 4 * nbr_h)])

    return body(idx5, table_pad)


def kernel(x, embeddings, W_e1, b_e1, W_e2, b_e2, W_q1, b_q1, W_q2, b_q2,
           W_d1, b_d1, W_d2, b_d2):
    e_flat = embeddings.reshape(K, LAT)
    table_pad = jnp.concatenate([e_flat, jnp.zeros((8, LAT), _F32)], axis=0)

    (z_e_t, z_dist_flat, k_blk, idx8, z_q, x_hat_q, x_hat_e) = _tc(
        x, e_flat, W_e1, b_e1.reshape(1, HID), W_e2, b_e2.reshape(1, LAT),
        W_q1, b_q1.reshape(1, HID), W_q2, b_q2.reshape(1, INP),
        W_d1, b_d1.reshape(1, HID), W_d2, b_d2.reshape(1, INP))
    z_e = z_e_t.T
    k = k_blk.reshape(B)
    idx5 = idx8[:, :5].reshape(B * 5)

    nbr = _sc_gather(idx5, table_pad)
    z_q_neighbors = nbr.reshape(B, 5, LAT)

    return (x_hat_q, x_hat_e, z_e, z_q, k, z_dist_flat, z_q_neighbors)
